# baseline (speedup 1.0000x reference)
import functools

import jax
import jax.numpy as jnp
from jax import lax
from jax.experimental import pallas as pl
from jax.experimental.pallas import tpu as pltpu
from jax.experimental.pallas import tpu_sc as plsc

NC = 2
NS = 16
NW = NC * NS
R = 4
L = 16


def _sig_body(x_ref, o_ref):
    s = jax.nn.sigmoid(x_ref[...])
    o_ref[...] = jnp.concatenate([s, s], axis=1)


def _make_sc(batch, width, emb):
    bpw = batch // NW
    nchunk = bpw // R
    egroups = emb // L
    gwidth = width // (L * egroups)

    mesh = plsc.VectorSubcoreMesh(
        core_axis_name="c", subcore_axis_name="s", num_cores=NC, num_subcores=NS
    )

    @functools.partial(
        pl.kernel,
        out_type=jax.ShapeDtypeStruct((batch, width), jnp.float32),
        mesh=mesh,
        scratch_types=[
            pltpu.VMEM((bpw // 128, 128), jnp.int32),
            pltpu.VMEM((bpw, 2 * emb), jnp.float32),
            pltpu.VMEM((2, R, width), jnp.float32),
            pltpu.VMEM((2, R, width), jnp.float32),
            pltpu.SemaphoreType.DMA,
            pltpu.SemaphoreType.DMA((2,)),
            pltpu.SemaphoreType.DMA((2,)),
        ],
    )
    def sc_kernel(
        table_hbm, rel_hbm, node_hbm, out_hbm,
        idx_v, scale_v, in_buf, out_buf, gsem, in_sems, out_sems,
    ):
        wid = lax.axis_index("s") * NC + lax.axis_index("c")
        gbase = wid * bpw

        pltpu.sync_copy(rel_hbm.at[pl.ds(wid * (bpw // 128), bpw // 128)], idx_v)
        for j in range(bpw // 128):
            pltpu.make_async_copy(
                table_hbm.at[idx_v.at[j]],
                scale_v.at[pl.ds(j * 128, 128)],
                gsem,
            ).start()
        for j in range(bpw // 128):
            pltpu.make_async_copy(
                table_hbm.at[idx_v.at[j]],
                scale_v.at[pl.ds(j * 128, 128)],
                gsem,
            ).wait()

        def in_copy(k, b):
            return pltpu.make_async_copy(
                node_hbm.at[pl.ds(gbase + k * R, R)], in_buf.at[b], in_sems.at[b]
            )

        def out_copy(k, b):
            return pltpu.make_async_copy(
                out_buf.at[b], out_hbm.at[pl.ds(gbase + k * R, R)], out_sems.at[b]
            )

        def compute(b, k):
            svs = []
            for r in range(R):
                row = k * R + r
                svs.append(
                    [scale_v[row, pl.ds(L * e, L)] for e in range(egroups)]
                )

            def gbody(g, carry):
                for r in range(R):
                    for e in range(egroups):
                        off = g * (L * egroups) + L * e
                        out_buf[b, r, pl.ds(off, L)] = (
                            in_buf[b, r, pl.ds(off, L)] * svs[r][e]
                        )
                return carry

            lax.fori_loop(0, gwidth, gbody, 0)

        in_copy(0, 0).start()
        in_copy(1, 1).start()

        def outer(c2, carry):
            for b in range(2):
                k = 2 * c2 + b

                in_copy(k, b).wait()

                @pl.when(c2 >= 1)
                def _():
                    out_copy(k - 2, b).wait()

                compute(b, k)
                out_copy(k, b).start()

                @pl.when(c2 < nchunk // 2 - 1)
                def _():
                    in_copy(k + 2, b).start()

            return carry

        lax.fori_loop(0, nchunk // 2, outer, 0)
        out_copy(nchunk - 2, 0).wait()
        out_copy(nchunk - 1, 1).wait()

    return sc_kernel


def kernel(node_emb, relation, rela_emb_):
    batch, hist, emb = node_emb.shape
    width = hist * emb
    sig = pl.pallas_call(
        _sig_body,
        out_shape=jax.ShapeDtypeStruct(
            (rela_emb_.shape[0], 2 * emb), rela_emb_.dtype
        ),
    )(rela_emb_)
    rel2 = relation.astype(jnp.int32).reshape(-1, 128)
    node2d = node_emb.reshape(batch, width)
    out2d = _make_sc(batch, width, emb)(sig, rel2, node2d)
    return out2d.reshape(batch, hist, emb)


# speedup vs baseline: 1.4649x; 1.4649x over previous
import functools

import jax
import jax.numpy as jnp
from jax import lax
from jax.experimental import pallas as pl
from jax.experimental.pallas import tpu as pltpu
from jax.experimental.pallas import tpu_sc as plsc

NC = 2
NS = 16
NW = NC * NS
CH = 512
NBUF = 3


def _sig_body(x_ref, o_ref):
    s = jax.nn.sigmoid(x_ref[...])
    o_ref[...] = jnp.concatenate([s, s], axis=1)


def _make_sc_gather(batch, emb2):
    bpw = batch // NW
    nidx = bpw // 128

    mesh = plsc.VectorSubcoreMesh(
        core_axis_name="c", subcore_axis_name="s", num_cores=NC, num_subcores=NS
    )

    @functools.partial(
        pl.kernel,
        out_type=jax.ShapeDtypeStruct((batch, emb2), jnp.float32),
        mesh=mesh,
        scratch_types=[
            pltpu.VMEM((nidx, 128), jnp.int32),
            pltpu.VMEM((bpw, emb2), jnp.float32),
            pltpu.SemaphoreType.DMA,
        ],
    )
    def sc_gather(table_hbm, rel_hbm, out_hbm, idx_v, scale_v, gsem):
        wid = lax.axis_index("s") * NC + lax.axis_index("c")
        pltpu.sync_copy(rel_hbm.at[pl.ds(wid * nidx, nidx)], idx_v)
        for j in range(nidx):
            pltpu.make_async_copy(
                table_hbm.at[idx_v.at[j]],
                scale_v.at[pl.ds(j * 128, 128)],
                gsem,
            ).start()
        for j in range(nidx):
            pltpu.make_async_copy(
                table_hbm.at[idx_v.at[j]],
                scale_v.at[pl.ds(j * 128, 128)],
                gsem,
            ).wait()
        pltpu.sync_copy(scale_v, out_hbm.at[pl.ds(wid * bpw, bpw)])

    return sc_gather


def _stream_body(scale_ref, node_hbm, out_hbm, node_buf, out_buf, in_sems, out_sems):
    nchunks = node_hbm.shape[0] // CH
    width = node_buf.shape[-1]

    def in_copy(k):
        s = k % NBUF
        return pltpu.make_async_copy(
            node_hbm.at[pl.ds(k * CH, CH)], node_buf.at[s], in_sems.at[s]
        )

    def out_copy(k):
        s = k % NBUF
        return pltpu.make_async_copy(
            out_buf.at[s], out_hbm.at[pl.ds(k * CH, CH)], out_sems.at[s]
        )

    for k in range(min(NBUF, nchunks)):
        in_copy(k).start()

    for k in range(nchunks):
        s = k % NBUF
        in_copy(k).wait()
        if k >= NBUF:
            out_copy(k - NBUF).wait()
        r2 = scale_ref[pl.ds(k * CH, CH), :]
        for j in range(width // 128):
            sl = slice(128 * j, 128 * (j + 1))
            out_buf[s, :, sl] = node_buf[s, :, sl] * r2
        out_copy(k).start()
        if k + NBUF < nchunks:
            in_copy(k + NBUF).start()

    for k in range(max(0, nchunks - NBUF), nchunks):
        out_copy(k).wait()


def kernel(node_emb, relation, rela_emb_):
    batch, hist, emb = node_emb.shape
    width = hist * emb
    sig = pl.pallas_call(
        _sig_body,
        out_shape=jax.ShapeDtypeStruct((rela_emb_.shape[0], 2 * emb), rela_emb_.dtype),
    )(rela_emb_)
    rel2 = relation.astype(jnp.int32).reshape(-1, 128)
    scale2 = _make_sc_gather(batch, 2 * emb)(sig, rel2)
    node2d = node_emb.reshape(batch, width)
    out2d = pl.pallas_call(
        _stream_body,
        in_specs=[
            pl.BlockSpec(memory_space=pltpu.MemorySpace.VMEM),
            pl.BlockSpec(memory_space=pltpu.MemorySpace.HBM),
        ],
        out_specs=pl.BlockSpec(memory_space=pltpu.MemorySpace.HBM),
        out_shape=jax.ShapeDtypeStruct((batch, width), node_emb.dtype),
        scratch_shapes=[
            pltpu.VMEM((NBUF, CH, width), jnp.float32),
            pltpu.VMEM((NBUF, CH, width), jnp.float32),
            pltpu.SemaphoreType.DMA((NBUF,)),
            pltpu.SemaphoreType.DMA((NBUF,)),
        ],
    )(scale2, node2d)
    return out2d.reshape(batch, hist, emb)
